# baseline (speedup 1.0000x reference)
import jax
import jax.numpy as jnp
from jax.experimental import pallas as pl
from jax.experimental.pallas import tpu as pltpu

_TABLE = 5
_GRID = 16


def _gather_body(idx_ref, out_ref):
    x = idx_ref[...]
    acc = jnp.zeros_like(x)
    for k in range(1, _TABLE):
        acc = jnp.where(x == k, jnp.int32(k), acc)
    out_ref[...] = acc


def kernel(indices, state):
    n = indices.size
    rows = n // 128
    flat = indices.reshape(rows, 128)
    br = rows // _GRID
    out = pl.pallas_call(
        _gather_body,
        grid=(_GRID,),
        in_specs=[pl.BlockSpec((br, 128), lambda i: (i, 0))],
        out_specs=pl.BlockSpec((br, 128), lambda i: (i, 0)),
        out_shape=jax.ShapeDtypeStruct((rows, 128), jnp.int32),
    )(flat)
    return out.reshape(indices.shape), state


# speedup vs baseline: 2.2765x; 2.2765x over previous
import jax
import jax.numpy as jnp
from jax.experimental import pallas as pl
from jax.experimental.pallas import tpu as pltpu

_TABLE = 5
_GRID = 16


def _gather_body(idx_ref, out_ref):
    x = idx_ref[...]
    acc = jnp.zeros_like(x)
    for k in range(1, _TABLE):
        acc = jnp.where(x == k, jnp.int32(k), acc)
    out_ref[...] = acc


def kernel(indices, state):
    rows, cols = indices.shape
    br = rows // _GRID
    out = pl.pallas_call(
        _gather_body,
        grid=(_GRID,),
        in_specs=[pl.BlockSpec((br, cols), lambda i: (i, 0))],
        out_specs=pl.BlockSpec((br, cols), lambda i: (i, 0)),
        out_shape=jax.ShapeDtypeStruct((rows, cols), jnp.int32),
    )(indices)
    return out, state


# speedup vs baseline: 2.3232x; 1.0205x over previous
import jax
import jax.numpy as jnp
from jax.experimental import pallas as pl
from jax.experimental.pallas import tpu as pltpu

_TABLE = 5
_GRID = 16


def _gather_body(idx_ref, out_ref):
    out_ref[...] = jnp.clip(idx_ref[...], 0, _TABLE - 1)


def kernel(indices, state):
    rows, cols = indices.shape
    br = rows // _GRID
    out = pl.pallas_call(
        _gather_body,
        grid=(_GRID,),
        in_specs=[pl.BlockSpec((br, cols), lambda i: (i, 0))],
        out_specs=pl.BlockSpec((br, cols), lambda i: (i, 0)),
        out_shape=jax.ShapeDtypeStruct((rows, cols), jnp.int32),
    )(indices)
    return out, state


# speedup vs baseline: 2.5444x; 1.0952x over previous
import jax
import jax.numpy as jnp
from jax.experimental import pallas as pl
from jax.experimental.pallas import tpu as pltpu

_TABLE = 5
_GRID = 8


def _gather_body(idx_ref, out_ref):
    out_ref[...] = jnp.clip(idx_ref[...], 0, _TABLE - 1)


def kernel(indices, state):
    rows, cols = indices.shape
    br = rows // _GRID
    out = pl.pallas_call(
        _gather_body,
        grid=(_GRID,),
        in_specs=[pl.BlockSpec((br, cols), lambda i: (i, 0))],
        out_specs=pl.BlockSpec((br, cols), lambda i: (i, 0)),
        out_shape=jax.ShapeDtypeStruct((rows, cols), jnp.int32),
    )(indices)
    return out, state


# speedup vs baseline: 2.5931x; 1.0191x over previous
import jax
import jax.numpy as jnp
from jax.experimental import pallas as pl
from jax.experimental.pallas import tpu as pltpu

_TABLE = 5
_GRID = 4


def _gather_body(idx_ref, out_ref):
    out_ref[...] = jnp.clip(idx_ref[...], 0, _TABLE - 1)


def kernel(indices, state):
    rows, cols = indices.shape
    br = rows // _GRID
    out = pl.pallas_call(
        _gather_body,
        grid=(_GRID,),
        in_specs=[pl.BlockSpec((br, cols), lambda i: (i, 0))],
        out_specs=pl.BlockSpec((br, cols), lambda i: (i, 0)),
        out_shape=jax.ShapeDtypeStruct((rows, cols), jnp.int32),
    )(indices)
    return out, state


# speedup vs baseline: 2.6926x; 1.0384x over previous
import jax
import jax.numpy as jnp
from jax.experimental import pallas as pl
from jax.experimental.pallas import tpu as pltpu

_TABLE = 5
_NBUF = 8
_CHUNK_ROWS = 1024


def _stream_body(idx_hbm, out_hbm, ibufs, obufs, sin, sout):
    rows = idx_hbm.shape[0]
    nchunks = rows // _CHUNK_ROWS

    def in_cp(c, b):
        return pltpu.make_async_copy(
            idx_hbm.at[pl.ds(c * _CHUNK_ROWS, _CHUNK_ROWS), :],
            ibufs.at[b], sin.at[b])

    def out_cp(c, b):
        return pltpu.make_async_copy(
            obufs.at[b],
            out_hbm.at[pl.ds(c * _CHUNK_ROWS, _CHUNK_ROWS), :],
            sout.at[b])

    for c in range(min(_NBUF, nchunks)):
        in_cp(c, c % _NBUF).start()
    for c in range(nchunks):
        b = c % _NBUF
        in_cp(c, b).wait()
        if c >= _NBUF:
            out_cp(c - _NBUF, b).wait()
        obufs[b] = jnp.clip(ibufs[b], 0, _TABLE - 1)
        out_cp(c, b).start()
        if c + _NBUF < nchunks:
            in_cp(c + _NBUF, b).start()
    for c in range(max(nchunks - _NBUF, 0), nchunks):
        out_cp(c, c % _NBUF).wait()


def kernel(indices, state):
    rows, cols = indices.shape
    out = pl.pallas_call(
        _stream_body,
        in_specs=[pl.BlockSpec(memory_space=pl.ANY)],
        out_specs=pl.BlockSpec(memory_space=pl.ANY),
        out_shape=jax.ShapeDtypeStruct((rows, cols), jnp.int32),
        scratch_shapes=[
            pltpu.VMEM((_NBUF, _CHUNK_ROWS, cols), jnp.int32),
            pltpu.VMEM((_NBUF, _CHUNK_ROWS, cols), jnp.int32),
            pltpu.SemaphoreType.DMA((_NBUF,)),
            pltpu.SemaphoreType.DMA((_NBUF,)),
        ],
    )(indices)
    return out, state
